# Initial kernel scaffold; baseline (speedup 1.0000x reference)
import jax
import jax.numpy as jnp
from jax.experimental import pallas as pl


def kernel(x, embedding):
    raise NotImplementedError("write your pallas kernel here")


# speedup vs baseline: 1.9041x; 1.9041x over previous
import jax
import jax.numpy as jnp
from jax.experimental import pallas as pl

B, C, HW = 16, 256, 1024
M = 1024
D = 256
T = B * HW


def _vq_body(x_ref, e_ref, et_ref, out_ref, idx_ref, minsum_ref, counts_ref):
    b = pl.program_id(0)
    xb = x_ref[...]
    emb = e_ref[...]
    cross = jnp.dot(emb, xb, preferred_element_type=jnp.float32)
    e_sq = jnp.sum(emb * emb, axis=1, keepdims=True)
    x_sq = jnp.sum(xb * xb, axis=0, keepdims=True)
    dist = (e_sq + x_sq) - 2.0 * cross
    minv = jnp.min(dist, axis=0, keepdims=True)
    miota = jax.lax.broadcasted_iota(jnp.int32, (M, HW), 0)
    idx = jnp.min(jnp.where(dist == minv, miota, jnp.int32(2**30)),
                  axis=0, keepdims=True)
    idx_ref[...] = idx
    minsum_ref[...] = jnp.sum(minv).reshape(1, 1)
    one_hot_t = jnp.where(miota == idx, 1.0, 0.0).astype(jnp.float32)
    out_ref[...] = jnp.dot(et_ref[...], one_hot_t,
                           preferred_element_type=jnp.float32,
                           precision=jax.lax.Precision.HIGHEST)
    cnt = jnp.sum(one_hot_t, axis=1, keepdims=True)

    @pl.when(b == 0)
    def _():
        counts_ref[...] = cnt

    @pl.when(b != 0)
    def _():
        counts_ref[...] += cnt


def kernel(x, embedding):
    x3 = x.reshape(B, C, HW)
    emb = embedding.reshape(M, D)
    embt = emb.T
    out_t, idx3, minsum, counts = pl.pallas_call(
        _vq_body,
        grid=(B,),
        in_specs=[
            pl.BlockSpec((None, C, HW), lambda b: (b, 0, 0)),
            pl.BlockSpec((M, D), lambda b: (0, 0)),
            pl.BlockSpec((D, M), lambda b: (0, 0)),
        ],
        out_specs=[
            pl.BlockSpec((None, D, HW), lambda b: (b, 0, 0)),
            pl.BlockSpec((None, 1, HW), lambda b: (b, 0, 0)),
            pl.BlockSpec((None, 1, 1), lambda b: (b, 0, 0)),
            pl.BlockSpec((M, 1), lambda b: (0, 0)),
        ],
        out_shape=[
            jax.ShapeDtypeStruct((B, D, HW), jnp.float32),
            jax.ShapeDtypeStruct((B, 1, HW), jnp.int32),
            jax.ShapeDtypeStruct((B, 1, 1), jnp.float32),
            jax.ShapeDtypeStruct((M, 1), jnp.float32),
        ],
    )(x3, emb, embt)
    out = out_t.reshape(B, C, 32, 32)
    indices = idx3.reshape(1, T)
    loss = 0.25 * (jnp.sum(minsum) / (T * D))
    avg_probs = counts.reshape(1, M) / T
    perplexity = jnp.exp(-jnp.sum(avg_probs * jnp.log(avg_probs + 1e-10),
                                  axis=-1))
    return (out, loss, jnp.sum(perplexity), indices)


# speedup vs baseline: 2.9698x; 1.5597x over previous
import jax
import jax.numpy as jnp
from jax.experimental import pallas as pl

B, C, HW = 16, 256, 1024
M = 1024
D = 256
T = B * HW


def _vq_body(x_ref, e_ref, et_ref, out_ref, idx_ref, minsum_ref, counts_ref):
    b = pl.program_id(0)
    xb = x_ref[...]
    emb = e_ref[...]
    cross = jnp.dot(emb, xb, preferred_element_type=jnp.float32)
    e_sq = jnp.sum(emb * emb, axis=1, keepdims=True)
    x_sq = jnp.sum(xb * xb, axis=0, keepdims=True)
    dist = (e_sq + x_sq) - 2.0 * cross
    minv = jnp.min(dist, axis=0, keepdims=True)
    miota = jax.lax.broadcasted_iota(jnp.int32, (M, HW), 0)
    idx = jnp.min(jnp.where(dist == minv, miota, jnp.int32(2**30)),
                  axis=0, keepdims=True)
    idx_ref[...] = idx
    minsum_ref[...] = jnp.sum(minv).reshape(1, 1)
    one_hot_t = jnp.where(miota == idx, 1.0, 0.0).astype(jnp.bfloat16)
    out_ref[...] = jnp.dot(et_ref[...], one_hot_t,
                           preferred_element_type=jnp.float32)
    cnt = jnp.sum(one_hot_t.astype(jnp.float32), axis=1, keepdims=True)

    @pl.when(b == 0)
    def _():
        counts_ref[...] = cnt

    @pl.when(b != 0)
    def _():
        counts_ref[...] += cnt


def kernel(x, embedding):
    x3 = x.reshape(B, C, HW)
    emb = embedding.reshape(M, D)
    embt = emb.T.astype(jnp.bfloat16)
    out_t, idx3, minsum, counts = pl.pallas_call(
        _vq_body,
        grid=(B,),
        in_specs=[
            pl.BlockSpec((None, C, HW), lambda b: (b, 0, 0)),
            pl.BlockSpec((M, D), lambda b: (0, 0)),
            pl.BlockSpec((D, M), lambda b: (0, 0)),
        ],
        out_specs=[
            pl.BlockSpec((None, D, HW), lambda b: (b, 0, 0)),
            pl.BlockSpec((None, 1, HW), lambda b: (b, 0, 0)),
            pl.BlockSpec((None, 1, 1), lambda b: (b, 0, 0)),
            pl.BlockSpec((M, 1), lambda b: (0, 0)),
        ],
        out_shape=[
            jax.ShapeDtypeStruct((B, D, HW), jnp.float32),
            jax.ShapeDtypeStruct((B, 1, HW), jnp.int32),
            jax.ShapeDtypeStruct((B, 1, 1), jnp.float32),
            jax.ShapeDtypeStruct((M, 1), jnp.float32),
        ],
    )(x3, emb, embt)
    out = out_t.reshape(B, C, 32, 32)
    indices = idx3.reshape(1, T)
    loss = 0.25 * (jnp.sum(minsum) / (T * D))
    avg_probs = counts.reshape(1, M) / T
    perplexity = jnp.exp(-jnp.sum(avg_probs * jnp.log(avg_probs + 1e-10),
                                  axis=-1))
    return (out, loss, jnp.sum(perplexity), indices)


# speedup vs baseline: 3.0471x; 1.0260x over previous
import jax
import jax.numpy as jnp
from jax.experimental import pallas as pl

B, C, HW = 16, 256, 1024
M = 1024
D = 256
T = B * HW


def _vq_body(x_ref, e_ref, et_ref, out_ref, idx_ref, minsum_ref, counts_ref):
    b = pl.program_id(0)
    xb = x_ref[...]
    emb = e_ref[...]
    cross_m2 = jnp.dot(-2.0 * emb, xb,
                       preferred_element_type=jnp.float32)
    e_sq = jnp.sum(emb * emb, axis=1, keepdims=True)
    x_sq = jnp.sum(xb * xb, axis=0, keepdims=True)
    dist = (e_sq + x_sq) + cross_m2
    minv = jnp.min(dist, axis=0, keepdims=True)
    miota = jax.lax.broadcasted_iota(jnp.int32, (M, HW), 0)
    idx = jnp.min(jnp.where(dist == minv, miota, jnp.int32(2**30)),
                  axis=0, keepdims=True)
    idx_ref[...] = idx
    minsum_ref[...] = jnp.sum(minv).reshape(1, 1)
    one_hot_t = jnp.where(miota == idx, 1.0, 0.0).astype(jnp.bfloat16)
    out_ref[...] = jnp.dot(et_ref[...], one_hot_t,
                           preferred_element_type=jnp.float32)
    cnt = jnp.sum(one_hot_t.astype(jnp.float32), axis=1, keepdims=True)

    @pl.when(b == 0)
    def _():
        counts_ref[...] = cnt

    @pl.when(b != 0)
    def _():
        counts_ref[...] += cnt


def kernel(x, embedding):
    x3 = x.reshape(B, C, HW)
    emb = embedding.reshape(M, D)
    embt = emb.T.astype(jnp.bfloat16)
    out_t, idx3, minsum, counts = pl.pallas_call(
        _vq_body,
        grid=(B,),
        in_specs=[
            pl.BlockSpec((None, C, HW), lambda b: (b, 0, 0)),
            pl.BlockSpec((M, D), lambda b: (0, 0)),
            pl.BlockSpec((D, M), lambda b: (0, 0)),
        ],
        out_specs=[
            pl.BlockSpec((None, D, HW), lambda b: (b, 0, 0)),
            pl.BlockSpec((None, 1, HW), lambda b: (b, 0, 0)),
            pl.BlockSpec((None, 1, 1), lambda b: (b, 0, 0)),
            pl.BlockSpec((M, 1), lambda b: (0, 0)),
        ],
        out_shape=[
            jax.ShapeDtypeStruct((B, D, HW), jnp.float32),
            jax.ShapeDtypeStruct((B, 1, HW), jnp.int32),
            jax.ShapeDtypeStruct((B, 1, 1), jnp.float32),
            jax.ShapeDtypeStruct((M, 1), jnp.float32),
        ],
    )(x3, emb, embt)
    out = out_t.reshape(B, C, 32, 32)
    indices = idx3.reshape(1, T)
    loss = 0.25 * (jnp.sum(minsum) / (T * D))
    avg_probs = counts.reshape(1, M) / T
    perplexity = jnp.exp(-jnp.sum(avg_probs * jnp.log(avg_probs + 1e-10),
                                  axis=-1))
    return (out, loss, jnp.sum(perplexity), indices)


# speedup vs baseline: 3.0529x; 1.0019x over previous
import jax
import jax.numpy as jnp
from jax.experimental import pallas as pl

B, C, HW = 16, 256, 1024
M = 1024
D = 256
T = B * HW


def _vq_body(x_ref, e_ref, ebf_ref, out_ref, idx_ref, minsum_ref, counts_ref):
    b = pl.program_id(0)
    xb = x_ref[...]
    emb = e_ref[...]
    cross_m2 = jnp.dot(-2.0 * emb, xb,
                       preferred_element_type=jnp.float32)
    e_sq = jnp.sum(emb * emb, axis=1, keepdims=True)
    x_sq = jnp.sum(xb * xb, axis=0, keepdims=True)
    dist = (e_sq + x_sq) + cross_m2
    minv = jnp.min(dist, axis=0, keepdims=True)
    miota = jax.lax.broadcasted_iota(jnp.int32, (M, HW), 0)
    idx = jnp.min(jnp.where(dist == minv, miota, jnp.int32(2**30)),
                  axis=0, keepdims=True)
    idx_ref[...] = idx
    minsum_ref[...] = jnp.sum(minv).reshape(1, 1)
    one_hot_t = jnp.where(miota == idx, 1.0, 0.0).astype(jnp.bfloat16)
    out_ref[...] = jax.lax.dot_general(
        ebf_ref[...], one_hot_t, (((0,), (0,)), ((), ())),
        preferred_element_type=jnp.float32)
    cnt = jnp.sum(one_hot_t.astype(jnp.float32), axis=1, keepdims=True)

    @pl.when(b == 0)
    def _():
        counts_ref[...] = cnt

    @pl.when(b != 0)
    def _():
        counts_ref[...] += cnt


def kernel(x, embedding):
    x3 = x.reshape(B, C, HW)
    emb = embedding.reshape(M, D)
    embf = emb.astype(jnp.bfloat16)
    out_t, idx3, minsum, counts = pl.pallas_call(
        _vq_body,
        grid=(B,),
        in_specs=[
            pl.BlockSpec((None, C, HW), lambda b: (b, 0, 0)),
            pl.BlockSpec((M, D), lambda b: (0, 0)),
            pl.BlockSpec((M, D), lambda b: (0, 0)),
        ],
        out_specs=[
            pl.BlockSpec((None, D, HW), lambda b: (b, 0, 0)),
            pl.BlockSpec((None, 1, HW), lambda b: (b, 0, 0)),
            pl.BlockSpec((None, 1, 1), lambda b: (b, 0, 0)),
            pl.BlockSpec((M, 1), lambda b: (0, 0)),
        ],
        out_shape=[
            jax.ShapeDtypeStruct((B, D, HW), jnp.float32),
            jax.ShapeDtypeStruct((B, 1, HW), jnp.int32),
            jax.ShapeDtypeStruct((B, 1, 1), jnp.float32),
            jax.ShapeDtypeStruct((M, 1), jnp.float32),
        ],
    )(x3, emb, embf)
    out = out_t.reshape(B, C, 32, 32)
    indices = idx3.reshape(1, T)
    loss = 0.25 * (jnp.sum(minsum) / (T * D))
    avg_probs = counts.reshape(1, M) / T
    perplexity = jnp.exp(-jnp.sum(avg_probs * jnp.log(avg_probs + 1e-10),
                                  axis=-1))
    return (out, loss, jnp.sum(perplexity), indices)


# speedup vs baseline: 3.2409x; 1.0616x over previous
import jax
import jax.numpy as jnp
from jax.experimental import pallas as pl
from jax.experimental.pallas import tpu as pltpu

B, C, HW = 16, 256, 1024
M = 1024
D = 256
T = B * HW


def _vq_body(x_ref, e_ref, ebf_ref, out_ref, idx_ref, loss_ref, perp_ref,
             cnt_acc, min_acc):
    b = pl.program_id(0)
    xb = x_ref[...]
    emb = e_ref[...]
    cross_m2 = jnp.dot(-2.0 * emb, xb,
                       preferred_element_type=jnp.float32)
    e_sq = jnp.sum(emb * emb, axis=1, keepdims=True)
    x_sq = jnp.sum(xb * xb, axis=0, keepdims=True)
    dist = (e_sq + x_sq) + cross_m2
    minv = jnp.min(dist, axis=0, keepdims=True)
    miota = jax.lax.broadcasted_iota(jnp.int32, (M, HW), 0)
    idx = jnp.min(jnp.where(dist == minv, miota, jnp.int32(2**30)),
                  axis=0, keepdims=True)
    idx_ref[...] = idx
    one_hot_t = jnp.where(miota == idx, 1.0, 0.0).astype(jnp.bfloat16)
    out_ref[...] = jax.lax.dot_general(
        ebf_ref[...], one_hot_t, (((0,), (0,)), ((), ())),
        preferred_element_type=jnp.float32)
    cnt = jnp.sum(one_hot_t.astype(jnp.float32), axis=1, keepdims=True)
    msum = jnp.sum(minv).reshape(1, 1)

    @pl.when(b == 0)
    def _():
        cnt_acc[...] = cnt
        min_acc[...] = msum

    @pl.when(b != 0)
    def _():
        cnt_acc[...] += cnt
        min_acc[...] += msum

    @pl.when(b == B - 1)
    def _():
        loss_ref[...] = (0.25 / (T * D)) * min_acc[...]
        p = cnt_acc[...] * (1.0 / T)
        ent = jnp.sum(p * jnp.log(p + 1e-10))
        perp_ref[...] = jnp.exp(-ent).reshape(1, 1)


def kernel(x, embedding):
    x3 = x.reshape(B, C, HW)
    emb = embedding.reshape(M, D)
    embf = emb.astype(jnp.bfloat16)
    out_t, idx3, loss2, perp2 = pl.pallas_call(
        _vq_body,
        grid=(B,),
        in_specs=[
            pl.BlockSpec((None, C, HW), lambda b: (b, 0, 0)),
            pl.BlockSpec((M, D), lambda b: (0, 0)),
            pl.BlockSpec((M, D), lambda b: (0, 0)),
        ],
        out_specs=[
            pl.BlockSpec((None, D, HW), lambda b: (b, 0, 0)),
            pl.BlockSpec((None, 1, HW), lambda b: (b, 0, 0)),
            pl.BlockSpec((1, 1), lambda b: (0, 0)),
            pl.BlockSpec((1, 1), lambda b: (0, 0)),
        ],
        out_shape=[
            jax.ShapeDtypeStruct((B, D, HW), jnp.float32),
            jax.ShapeDtypeStruct((B, 1, HW), jnp.int32),
            jax.ShapeDtypeStruct((1, 1), jnp.float32),
            jax.ShapeDtypeStruct((1, 1), jnp.float32),
        ],
        scratch_shapes=[
            pltpu.VMEM((M, 1), jnp.float32),
            pltpu.VMEM((1, 1), jnp.float32),
        ],
    )(x3, emb, embf)
    out = out_t.reshape(B, C, 32, 32)
    indices = idx3.reshape(1, T)
    return (out, loss2.reshape(()), perp2.reshape(()), indices)
